# baseline (speedup 1.0000x reference)
import functools

import jax
import jax.numpy as jnp
from jax import lax
from jax.experimental import pallas as pl
from jax.experimental.pallas import tpu as pltpu
from jax.experimental.pallas import tpu_sc as plsc

_R, _C = 128, 8192
_N = _R * _C
_NC, _NS, _L = 2, 16, 16
_NW = _NC * _NS
_PER_W = _N // _NW
_CHUNK = 8192
_NCHUNK = _PER_W // _CHUNK
_UNROLL = 8
_NACC = 4

_C0 = 2.996352304762695e-05
_C1 = 0.9995259642601013
_C2 = -0.50326007604599
_C3 = 0.354997843503952
_C4 = -0.2194514125585556
_LN2 = 0.6931471805599453
_EXP_REBASE = 0x3F3504F3


def _loss_vec(yv, tv):
    a = jnp.where(tv == 1.0, yv, 1.0 - yv)
    ib = lax.bitcast_convert_type(a, jnp.int32)
    e = (ib - _EXP_REBASE) >> 23
    m = lax.bitcast_convert_type(ib - (e << 23), jnp.float32)
    x = m - 1.0
    p = _C4 * x + _C3
    p = p * x + _C2
    p = p * x + _C1
    p = p * x + _C0
    loga = e.astype(jnp.float32) * _LN2 + p
    return (a - 1.0) * loga


def _make_sc_kernel():
    mesh = plsc.VectorSubcoreMesh(core_axis_name="c", subcore_axis_name="s")
    vec_per_chunk = _CHUNK // _L
    iters = vec_per_chunk // _UNROLL

    @functools.partial(
        pl.kernel,
        mesh=mesh,
        out_type=jax.ShapeDtypeStruct((_NW * _L,), jnp.float32),
        scratch_types=[
            pltpu.VMEM((_CHUNK,), jnp.float32),
            pltpu.VMEM((_CHUNK,), jnp.float32),
            pltpu.VMEM((_CHUNK,), jnp.float32),
            pltpu.VMEM((_CHUNK,), jnp.float32),
            pltpu.VMEM((_L,), jnp.float32),
            pltpu.SemaphoreType.DMA,
            pltpu.SemaphoreType.DMA,
        ],
    )
    def _sc(y_hbm, t_hbm, out_hbm, yb0, yb1, tb0, tb1, accb, sem0, sem1):
        wid = lax.axis_index("s") * _NC + lax.axis_index("c")
        base = wid * _PER_W
        ybufs = (yb0, yb1)
        tbufs = (tb0, tb1)
        sems = (sem0, sem1)

        def start(g, slot):
            off = base + g * _CHUNK
            cy = pltpu.make_async_copy(
                y_hbm.at[pl.ds(off, _CHUNK)], ybufs[slot], sems[slot])
            ct = pltpu.make_async_copy(
                t_hbm.at[pl.ds(off, _CHUNK)], tbufs[slot], sems[slot])
            cy.start()
            ct.start()
            return cy, ct

        def compute(slot, accs):
            yb, tb = ybufs[slot], tbufs[slot]

            def body(i, accs):
                base_v = i * (_UNROLL * _L)
                new = list(accs)
                for u in range(_UNROLL):
                    off = base_v + u * _L
                    yv = yb[pl.ds(off, _L)]
                    tv = tb[pl.ds(off, _L)]
                    new[u % _NACC] = new[u % _NACC] + _loss_vec(yv, tv)
                return tuple(new)

            return lax.fori_loop(0, iters, body, accs)

        accs = tuple(jnp.zeros((_L,), jnp.float32) for _ in range(_NACC))
        inflight = [start(0, 0), start(1, 1)]
        for g in range(_NCHUNK):
            slot = g % 2
            for c in inflight[g]:
                c.wait()
            accs = compute(slot, accs)
            if g + 2 < _NCHUNK:
                inflight.append(start(g + 2, slot))

        total = accs[0] + accs[1] + accs[2] + accs[3]
        accb[...] = total
        pltpu.sync_copy(accb, out_hbm.at[pl.ds(wid * _L, _L)])

    return _sc


_sc_kernel = _make_sc_kernel()


def kernel(y, target):
    part = _sc_kernel(y.reshape(_N), target.reshape(_N))
    return jnp.sum(part) / _N


# speedup vs baseline: 1.1430x; 1.1430x over previous
import functools

import jax
import jax.numpy as jnp
from jax import lax
from jax.experimental import pallas as pl
from jax.experimental.pallas import tpu as pltpu
from jax.experimental.pallas import tpu_sc as plsc

_R, _C = 128, 8192
_N = _R * _C
_NC, _NS, _L = 2, 16, 16
_NW = _NC * _NS
_ROWS_W = _R // _NW
_UNROLL = 8
_NACC = 4

_C0 = -1.82562255859375
_C1 = 2.9512929916381836
_C2 = -1.4271800518035889
_C3 = 0.3017500042915344
_LN2 = 0.6931471805599453
_EXP_REBASE = 0x3F3504F3


def _loss_vec(yv, tv):
    a = jnp.where(tv == 1.0, yv, 1.0 - yv)
    ib = lax.bitcast_convert_type(a, jnp.int32)
    e = (ib - _EXP_REBASE) >> 23
    m = lax.bitcast_convert_type(ib - (e << 23), jnp.float32)
    p = _C3 * m + _C2
    p = p * m + _C1
    p = p * m + _C0
    loga = e.astype(jnp.float32) * _LN2 + p
    return (a - 1.0) * loga


def _make_sc_kernel():
    mesh = plsc.VectorSubcoreMesh(core_axis_name="c", subcore_axis_name="s")
    iters = _ROWS_W * _C // (_UNROLL * _L)

    @functools.partial(
        pl.kernel,
        mesh=mesh,
        out_type=jax.ShapeDtypeStruct((_NW * _L,), jnp.float32),
        scratch_types=[
            pltpu.VMEM((_ROWS_W, _C), jnp.float32),
            pltpu.VMEM((_ROWS_W, _C), jnp.float32),
            pltpu.VMEM((_L,), jnp.float32),
            pltpu.SemaphoreType.DMA,
        ],
    )
    def _sc(y_hbm, t_hbm, out_hbm, yb, tb, accb, sem):
        wid = lax.axis_index("s") * _NC + lax.axis_index("c")
        row0 = wid * _ROWS_W
        cy = pltpu.make_async_copy(y_hbm.at[pl.ds(row0, _ROWS_W)], yb, sem)
        ct = pltpu.make_async_copy(t_hbm.at[pl.ds(row0, _ROWS_W)], tb, sem)
        cy.start()
        ct.start()
        cy.wait()
        ct.wait()

        iters_row = _C // (_UNROLL * _L)

        def body(i, accs):
            r = i // iters_row
            col = (i % iters_row) * (_UNROLL * _L)
            new = list(accs)
            for u in range(_UNROLL):
                off = col + u * _L
                yv = yb[r, pl.ds(off, _L)]
                tv = tb[r, pl.ds(off, _L)]
                new[u % _NACC] = new[u % _NACC] + _loss_vec(yv, tv)
            return tuple(new)

        accs = tuple(jnp.zeros((_L,), jnp.float32) for _ in range(_NACC))
        accs = lax.fori_loop(0, iters, body, accs)
        accb[...] = accs[0] + accs[1] + accs[2] + accs[3]
        pltpu.sync_copy(accb, out_hbm.at[pl.ds(wid * _L, _L)])

    return _sc


_sc_kernel = _make_sc_kernel()


def kernel(y, target):
    part = _sc_kernel(y, target)
    return jnp.sum(part) / _N
